# baseline (speedup 1.0000x reference)
import functools

import jax
import jax.numpy as jnp
from jax.experimental import pallas as pl

DIM = 128
J = 17
OFFSETS = (-7, -6, -4, -3, -1, 1, 4, 7)
TILE = 32
R = TILE * J


def _body(x_ref, att_ref, w_ref, b_ref, ls_ref, w1_ref, w2_ref,
          ctab_ref, sb_ref, bias_ref, out_ref, gf_ref):
    f32 = jnp.float32
    xb = x_ref[...]
    ones = jnp.full((DIM, DIM), 1.0 / DIM, f32)
    mu = jnp.dot(xb, ones, preferred_element_type=f32)
    xc = xb - mu
    var = jnp.dot(xc * xc, ones, preferred_element_type=f32)
    xn = xc * jax.lax.rsqrt(var + 1e-5) * w_ref[...] + b_ref[...]

    y = ctab_ref[0 * R:1 * R, :] * jnp.roll(xn, -OFFSETS[0], axis=0)
    for m in range(1, len(OFFSETS)):
        y = y + ctab_ref[m * R:(m + 1) * R, :] * jnp.roll(xn, -OFFSETS[m], axis=0)

    hi = jax.lax.Precision.HIGHEST
    h = (jnp.dot(xn, w1_ref[...], precision=hi, preferred_element_type=f32)
         + jnp.dot(y, w2_ref[...], precision=hi, preferred_element_type=f32))
    g = jnp.maximum(xn + h * sb_ref[...] + bias_ref[...], 0.0)
    xs = ls_ref[...] * g
    gf_ref[...] = 0.5 * xs
    out_ref[...] = xs + att_ref[...] + xb


@functools.partial(jax.jit, static_argnames=())
def kernel(x, attention_feat, norm1_w, norm1_b, ls1, U_w, U_b, V_w, V_b,
           bn_w, bn_b, adj):
    B, T, Jdim, D = x.shape
    N = B * T * Jdim
    x2 = x.reshape(N, D)
    att2 = attention_feat.reshape(N, D)

    deg = adj.sum(-1)
    dinv = deg ** -0.5
    na = dinv[:, None] * adj * dinv[None, :]
    cols = []
    for d in OFFSETS:
        diag = jnp.diagonal(na, offset=d)
        c17 = jnp.pad(diag, (0, d) if d > 0 else (-d, 0))
        crow = jnp.tile(c17, TILE)
        cols.append(jnp.broadcast_to(crow[:, None], (R, D)))
    ctab = jnp.concatenate(cols, axis=0)

    s = bn_w * (1.0 / jnp.sqrt(1.0 + 1e-5))
    rs = na.sum(-1)
    bias17 = (U_b[None, :] + rs[:, None] * V_b[None, :]) * s[:, None] \
        + bn_b[:, None]
    sbR = jnp.broadcast_to(jnp.tile(s, TILE)[:, None], (R, D))
    biasR = jnp.tile(bias17, (TILE, 1))

    w1 = U_w.T
    w2 = V_w.T
    lw = norm1_w.reshape(1, D)
    lb = norm1_b.reshape(1, D)
    ls = ls1.reshape(1, D)

    grid = (N // R,)
    row_spec = pl.BlockSpec((R, D), lambda i: (i, 0))
    const = lambda shape: pl.BlockSpec(shape, lambda i: (0, 0))
    out2, gf2 = pl.pallas_call(
        _body,
        grid=grid,
        in_specs=[
            row_spec,
            row_spec,
            const((1, D)),
            const((1, D)),
            const((1, D)),
            const((D, D)),
            const((D, D)),
            const((len(OFFSETS) * R, D)),
            const((R, D)),
            const((R, D)),
        ],
        out_specs=(row_spec, row_spec),
        out_shape=(jax.ShapeDtypeStruct((N, D), jnp.float32),
                   jax.ShapeDtypeStruct((N, D), jnp.float32)),
    )(x2, att2, lw, lb, ls, w1, w2, ctab, sbR, biasR)
    return (out2.reshape(B, T, Jdim, D), gf2.reshape(B, T, Jdim, D))
